# baseline (speedup 1.0000x reference)
import functools

import numpy as np
import jax
import jax.numpy as jnp
from jax import lax
from jax.experimental import pallas as pl
from jax.experimental.pallas import tpu as pltpu
from jax.experimental.pallas import tpu_sc as plsc


def _norm_pdf_np(z):
    return np.exp(-0.5 * z * z) / np.sqrt(2.0 * np.pi)


def _select_indices(frame_count: int) -> list:
    rng = np.random.RandomState(0)
    num_segments = 1
    idxs = np.linspace(0, frame_count - 1, frame_count, dtype=int)
    if frame_count <= num_segments * 2:
        idxs = np.repeat(idxs, int(frame_count * num_segments / len(idxs)))
        frame_count *= num_segments
    seg_sizes = _norm_pdf_np(np.linspace(-1, 1, num_segments))
    seg_sizes = 1 - seg_sizes if frame_count > num_segments else seg_sizes
    seg_sizes = seg_sizes / seg_sizes.sum() * frame_count
    seg_sizes = seg_sizes.astype(int)
    choices = []
    last_idx = 0
    for i, seg_size in enumerate(seg_sizes):
        next_idx = last_idx + seg_size if i < len(seg_sizes) - 1 else None
        choices.append(int(rng.choice(idxs[last_idx:next_idx], 1)[0]))
        last_idx = next_idx
    return choices


@functools.lru_cache(maxsize=None)
def _make_sc_gather(n_groups: int, ftiles: int, wsub: int, flanes: int,
                    ift: int, ilane: int):
    info = plsc.get_sparse_core_info()
    nw = info.num_cores * info.num_subcores
    while nw > 1 and (n_groups % nw != 0 or (n_groups // nw) % 8 != 0):
        nw -= 1
    chunk = n_groups // nw
    mesh = plsc.VectorSubcoreMesh(core_axis_name="c", subcore_axis_name="s")

    il8 = (ilane // 8) * 8
    lane_in_block = ilane - il8
    assert chunk % 16 == 0

    @functools.partial(
        pl.kernel,
        mesh=mesh,
        out_type=jax.ShapeDtypeStruct((n_groups, wsub), jnp.float32),
        scratch_types=[
            pltpu.VMEM((wsub, chunk, 8), jnp.float32),
            pltpu.VMEM((chunk, wsub), jnp.float32),
            pltpu.SemaphoreType.DMA,
        ],
        compiler_params=pltpu.CompilerParams(
            use_tc_tiling_on_sc=False,
            needs_layout_passes=False,
            skip_device_barrier=True,
        ),
    )
    def sc_gather(x_hbm, out_hbm, blocks, colt, sem):
        wid = lax.axis_index("s") * info.num_cores + lax.axis_index("c")
        g0 = wid * chunk

        @pl.when(wid < nw)
        def _():
            descs = [
                pltpu.make_async_copy(
                    x_hbm.at[pl.ds(g0, chunk), ift, wi, pl.ds(il8, 8)],
                    blocks.at[wi],
                    sem,
                )
                for wi in range(wsub)
            ]
            for d in descs:
                d.start()
            lane16 = lax.iota(jnp.int32, 16)
            lanei = jnp.full((16,), lane_in_block, jnp.int32)
            for wi in range(wsub):
                descs[wi].wait()
                coli = jnp.full((16,), wi, jnp.int32)
                for j in range(chunk // 16):
                    rows = j * 16 + lane16
                    vals = plsc.load_gather(blocks.at[wi], [rows, lanei])
                    plsc.store_scatter(colt, [rows, coli], vals)
            pltpu.sync_copy(colt, out_hbm.at[pl.ds(g0, chunk)])

    return sc_gather


@functools.lru_cache(maxsize=None)
def _make_tc_gather(n_tc: int, block_g: int, off_blocks: int, ift: int,
                    ilane: int, wsub: int, flanes: int):
    def body(x_ref, o_ref):
        b = x_ref[:, 0]
        onehot = (
            lax.broadcasted_iota(jnp.int32, (1, 1, flanes), 2) == ilane
        ).astype(jnp.float32)
        o_ref[...] = jnp.sum(b * onehot, axis=-1)

    return pl.pallas_call(
        body,
        out_shape=jax.ShapeDtypeStruct((n_tc, wsub), jnp.float32),
        grid=(n_tc // block_g,),
        in_specs=[
            pl.BlockSpec(
                (block_g, 1, wsub, flanes),
                lambda i: (off_blocks + i, ift, 0, 0),
            )
        ],
        out_specs=pl.BlockSpec((block_g, wsub), lambda i: (i, 0)),
    )


def kernel(x):
    frames, chn, hgt, wdt = x.shape
    choices = _select_indices(frames)
    ch = choices[0]
    flanes = 128
    wsub = 8
    ftiles = frames // flanes
    wtiles = wdt // wsub
    n_groups = chn * hgt * wtiles
    ift, ilane = ch // flanes, ch % flanes
    xv = (
        x.reshape(ftiles, flanes, chn, hgt, wtiles, wsub)
        .transpose(2, 3, 4, 0, 5, 1)
        .reshape(n_groups, ftiles, wsub, flanes)
    )
    n_sc = n_groups // 2
    block_g = 1176
    sc_gather = _make_sc_gather(n_sc, ftiles, wsub, flanes, ift, ilane)
    tc_gather = _make_tc_gather(
        n_groups - n_sc, block_g, n_sc // block_g, ift, ilane, wsub, flanes
    )
    sc_part = sc_gather(xv)
    tc_part = tc_gather(xv)
    out2 = jnp.concatenate([sc_part, tc_part], axis=0)
    return out2.reshape(1, chn, hgt, wdt)


# speedup vs baseline: 1.8345x; 1.8345x over previous
import functools

import numpy as np
import jax
import jax.numpy as jnp
from jax import lax
from jax.experimental import pallas as pl
from jax.experimental.pallas import tpu as pltpu
from jax.experimental.pallas import tpu_sc as plsc


def _norm_pdf_np(z):
    return np.exp(-0.5 * z * z) / np.sqrt(2.0 * np.pi)


def _select_indices(frame_count: int) -> list:
    rng = np.random.RandomState(0)
    num_segments = 1
    idxs = np.linspace(0, frame_count - 1, frame_count, dtype=int)
    if frame_count <= num_segments * 2:
        idxs = np.repeat(idxs, int(frame_count * num_segments / len(idxs)))
        frame_count *= num_segments
    seg_sizes = _norm_pdf_np(np.linspace(-1, 1, num_segments))
    seg_sizes = 1 - seg_sizes if frame_count > num_segments else seg_sizes
    seg_sizes = seg_sizes / seg_sizes.sum() * frame_count
    seg_sizes = seg_sizes.astype(int)
    choices = []
    last_idx = 0
    for i, seg_size in enumerate(seg_sizes):
        next_idx = last_idx + seg_size if i < len(seg_sizes) - 1 else None
        choices.append(int(rng.choice(idxs[last_idx:next_idx], 1)[0]))
        last_idx = next_idx
    return choices


@functools.lru_cache(maxsize=None)
def _make_sc_gather(n_groups: int, ftiles: int, wsub: int, flanes: int,
                    ift: int, ilane: int):
    info = plsc.get_sparse_core_info()
    nw = info.num_cores * info.num_subcores
    while nw > 1 and n_groups % nw != 0:
        nw -= 1
    chunk = n_groups // nw
    mesh = plsc.VectorSubcoreMesh(core_axis_name="c", subcore_axis_name="s")

    il8 = (ilane // 8) * 8
    lane_in_block = ilane - il8
    full_vregs, tail = divmod(chunk, 16)

    @functools.partial(
        pl.kernel,
        mesh=mesh,
        out_type=jax.ShapeDtypeStruct((n_groups, wsub), jnp.float32),
        scratch_types=[
            pltpu.VMEM((wsub, chunk, 8), jnp.float32),
            pltpu.VMEM((chunk, wsub), jnp.float32),
            pltpu.SemaphoreType.DMA,
        ],
        compiler_params=pltpu.CompilerParams(
            use_tc_tiling_on_sc=False,
            needs_layout_passes=False,
            skip_device_barrier=True,
        ),
    )
    def sc_gather(x_hbm, out_hbm, blocks, colt, sem):
        wid = lax.axis_index("s") * info.num_cores + lax.axis_index("c")
        g0 = wid * chunk

        @pl.when(wid < nw)
        def _():
            descs = [
                pltpu.make_async_copy(
                    x_hbm.at[pl.ds(g0, chunk), ift, wi, pl.ds(il8, 8)],
                    blocks.at[wi],
                    sem,
                )
                for wi in range(wsub)
            ]
            for d in descs:
                d.start()
            lane16 = lax.iota(jnp.int32, 16)
            lanei = jnp.full((16,), lane_in_block, jnp.int32)
            for wi in range(wsub):
                descs[wi].wait()
                coli = jnp.full((16,), wi, jnp.int32)
                for j in range(full_vregs):
                    rows = j * 16 + lane16
                    vals = plsc.load_gather(blocks.at[wi], [rows, lanei])
                    plsc.store_scatter(colt, [rows, coli], vals)
                if tail:
                    rows = full_vregs * 16 + lane16
                    msk = lane16 < tail
                    vals = plsc.load_gather(
                        blocks.at[wi], [rows, lanei], mask=msk
                    )
                    plsc.store_scatter(colt, [rows, coli], vals, mask=msk)
            pltpu.sync_copy(colt, out_hbm.at[pl.ds(g0, chunk)])

    return sc_gather


def kernel(x):
    frames, chn, hgt, wdt = x.shape
    choices = _select_indices(frames)
    ch = choices[0]
    flanes = 128
    wsub = 8
    ftiles = frames // flanes
    wtiles = wdt // wsub
    n_groups = chn * hgt * wtiles
    xv = (
        x.reshape(ftiles, flanes, chn, hgt, wtiles, wsub)
        .transpose(2, 3, 4, 0, 5, 1)
        .reshape(n_groups, ftiles, wsub, flanes)
    )
    sc_gather = _make_sc_gather(
        n_groups, ftiles, wsub, flanes, ch // flanes, ch % flanes
    )
    out2 = sc_gather(xv)
    return out2.reshape(1, chn, hgt, wdt)


# speedup vs baseline: 1.8371x; 1.0014x over previous
import functools

import numpy as np
import jax
import jax.numpy as jnp
from jax import lax
from jax.experimental import pallas as pl
from jax.experimental.pallas import tpu as pltpu
from jax.experimental.pallas import tpu_sc as plsc


def _norm_pdf_np(z):
    return np.exp(-0.5 * z * z) / np.sqrt(2.0 * np.pi)


def _select_indices(frame_count: int) -> list:
    rng = np.random.RandomState(0)
    num_segments = 1
    idxs = np.linspace(0, frame_count - 1, frame_count, dtype=int)
    if frame_count <= num_segments * 2:
        idxs = np.repeat(idxs, int(frame_count * num_segments / len(idxs)))
        frame_count *= num_segments
    seg_sizes = _norm_pdf_np(np.linspace(-1, 1, num_segments))
    seg_sizes = 1 - seg_sizes if frame_count > num_segments else seg_sizes
    seg_sizes = seg_sizes / seg_sizes.sum() * frame_count
    seg_sizes = seg_sizes.astype(int)
    choices = []
    last_idx = 0
    for i, seg_size in enumerate(seg_sizes):
        next_idx = last_idx + seg_size if i < len(seg_sizes) - 1 else None
        choices.append(int(rng.choice(idxs[last_idx:next_idx], 1)[0]))
        last_idx = next_idx
    return choices


@functools.lru_cache(maxsize=None)
def _make_sc_gather(n_groups: int, ftiles: int, wsub: int, flanes: int,
                    ift: int, ilane: int):
    info = plsc.get_sparse_core_info()
    nw = info.num_cores * info.num_subcores
    while nw > 1 and n_groups % nw != 0:
        nw -= 1
    chunk = n_groups // nw
    mesh = plsc.VectorSubcoreMesh(core_axis_name="c", subcore_axis_name="s")

    il8 = (ilane // 8) * 8
    lane_in_block = ilane - il8
    full_vregs, tail = divmod(chunk, 16)

    @functools.partial(
        pl.kernel,
        mesh=mesh,
        out_type=jax.ShapeDtypeStruct((n_groups, wsub), jnp.float32),
        scratch_types=[
            pltpu.VMEM((wsub, chunk, 8), jnp.float32),
            pltpu.VMEM((chunk, wsub), jnp.float32),
            pltpu.SemaphoreType.DMA,
        ],
        compiler_params=pltpu.CompilerParams(
            use_tc_tiling_on_sc=False,
            needs_layout_passes=False,
        ),
    )
    def sc_gather(x_hbm, out_hbm, blocks, colt, sem):
        wid = lax.axis_index("s") * info.num_cores + lax.axis_index("c")
        g0 = wid * chunk

        @pl.when(wid < nw)
        def _():
            descs = [
                pltpu.make_async_copy(
                    x_hbm.at[pl.ds(g0, chunk), ift, wi, pl.ds(il8, 8)],
                    blocks.at[wi],
                    sem,
                )
                for wi in range(wsub)
            ]
            for d in descs:
                d.start()
            lane16 = lax.iota(jnp.int32, 16)
            lanei = jnp.full((16,), lane_in_block, jnp.int32)
            for wi in range(wsub):
                descs[wi].wait()
                coli = jnp.full((16,), wi, jnp.int32)
                for j in range(full_vregs):
                    rows = j * 16 + lane16
                    vals = plsc.load_gather(blocks.at[wi], [rows, lanei])
                    plsc.store_scatter(colt, [rows, coli], vals)
                if tail:
                    rows = full_vregs * 16 + lane16
                    msk = lane16 < tail
                    vals = plsc.load_gather(
                        blocks.at[wi], [rows, lanei], mask=msk
                    )
                    plsc.store_scatter(colt, [rows, coli], vals, mask=msk)
            pltpu.sync_copy(colt, out_hbm.at[pl.ds(g0, chunk)])

    return sc_gather


def kernel(x):
    frames, chn, hgt, wdt = x.shape
    choices = _select_indices(frames)
    ch = choices[0]
    flanes = 128
    wsub = 8
    ftiles = frames // flanes
    wtiles = wdt // wsub
    n_groups = chn * hgt * wtiles
    xv = (
        x.reshape(ftiles, flanes, chn, hgt, wtiles, wsub)
        .transpose(2, 3, 4, 0, 5, 1)
        .reshape(n_groups, ftiles, wsub, flanes)
    )
    sc_gather = _make_sc_gather(
        n_groups, ftiles, wsub, flanes, ch // flanes, ch % flanes
    )
    out2 = sc_gather(xv)
    return out2.reshape(1, chn, hgt, wdt)
